# baseline (device time: 34221 ns/iter reference)
import jax
import jax.numpy as jnp
from jax import lax
from jax.experimental import pallas as pl
from jax.experimental.pallas import tpu as pltpu

N_DEV = 16
B, S, C, H = 4, 512, 256, 256
ROWS = B * S
CHUNK = ROWS // N_DEV
HALF = CHUNK // 2


def kernel(x, k, Wp):
    def body(x_ref, k_ref, w_ref, out_ref,
             acc_ref, temp_ref, final_ref,
             rs_send, rs_recv, ag_send, ag_recv):
        d = lax.axis_index("i")

        bsem = pltpu.get_barrier_semaphore()
        for o in range(1, N_DEV):
            pl.semaphore_signal(
                bsem, inc=1,
                device_id=(jnp.mod(d + o, N_DEV),),
                device_id_type=pl.DeviceIdType.MESH,
            )

        xv = x_ref[:, :, :]
        kv = k_ref[:, :]
        conv = xv * kv[3:4, :][None, :, :]
        for t in range(3):
            sh = 3 - t
            shifted = jnp.concatenate(
                [jnp.zeros((B, sh, C), jnp.float32), xv[:, : S - sh, :]],
                axis=1,
            )
            conv += shifted * kv[t:t + 1, :][None, :, :]
        a = conv * jax.nn.sigmoid(conv)
        partial = jnp.dot(
            a.reshape(ROWS, C).astype(jnp.bfloat16),
            w_ref[:, :].astype(jnp.bfloat16),
            preferred_element_type=jnp.float32,
        )
        acc_ref[...] = partial.reshape(N_DEV, 2, HALF, H).astype(jnp.bfloat16)

        pl.semaphore_wait(bsem, N_DEV - 1)

        p1 = {}
        for h in range(2):
            for o in range(1, N_DEV):
                tgt = jnp.mod(d + o, N_DEV)
                r = pltpu.make_async_remote_copy(
                    src_ref=acc_ref.at[tgt, h],
                    dst_ref=temp_ref.at[h, o],
                    send_sem=rs_send.at[h, o],
                    recv_sem=rs_recv.at[h, o],
                    device_id=(tgt,),
                    device_id_type=pl.DeviceIdType.MESH,
                )
                r.start()
                p1[(h, o)] = r

        p2 = {}
        for h in range(2):
            for o in range(1, N_DEV):
                p1[(h, o)].wait_recv()
            own = acc_ref[pl.ds(d, 1), h, :, :].astype(jnp.float32)
            others = jnp.sum(temp_ref[h, 1:, :, :].astype(jnp.float32), axis=0)
            reduced = (own[0] + others).astype(jnp.bfloat16)
            final_ref[pl.ds(d, 1), h, :, :] = reduced[None, :, :]
            for o in range(1, N_DEV):
                tgt = jnp.mod(d + o, N_DEV)
                r = pltpu.make_async_remote_copy(
                    src_ref=final_ref.at[d, h],
                    dst_ref=final_ref.at[d, h],
                    send_sem=ag_send.at[h, o],
                    recv_sem=ag_recv.at[h, o],
                    device_id=(tgt,),
                    device_id_type=pl.DeviceIdType.MESH,
                )
                r.start()
                p2[(h, o)] = r

        for h in range(2):
            for o in range(1, N_DEV):
                p2[(h, o)].wait_recv()

        out_ref[...] = (
            final_ref[:, :, :, :].astype(jnp.float32).reshape(B, S, H)
        )

        for r in p1.values():
            r.wait_send()
        for r in p2.values():
            r.wait_send()

    return pl.pallas_call(
        body,
        out_shape=jax.ShapeDtypeStruct((B, S, H), jnp.float32),
        in_specs=[pl.BlockSpec(memory_space=pltpu.VMEM)] * 3,
        out_specs=pl.BlockSpec(memory_space=pltpu.VMEM),
        scratch_shapes=[
            pltpu.VMEM((N_DEV, 2, HALF, H), jnp.bfloat16),
            pltpu.VMEM((2, N_DEV, HALF, H), jnp.bfloat16),
            pltpu.VMEM((N_DEV, 2, HALF, H), jnp.bfloat16),
            pltpu.SemaphoreType.DMA((2, N_DEV)),
            pltpu.SemaphoreType.DMA((2, N_DEV)),
            pltpu.SemaphoreType.DMA((2, N_DEV)),
            pltpu.SemaphoreType.DMA((2, N_DEV)),
        ],
        compiler_params=pltpu.CompilerParams(collective_id=0),
    )(x, k, Wp)


# device time: 32386 ns/iter; 1.0567x vs baseline; 1.0567x over previous
import jax
import jax.numpy as jnp
from jax import lax
from jax.experimental import pallas as pl
from jax.experimental.pallas import tpu as pltpu

N_DEV = 16
B, S, C, H = 4, 512, 256, 256
ROWS = B * S
CHUNK = ROWS // N_DEV


def kernel(x, k, Wp):
    def body(x_ref, k_ref, w_ref, out_ref,
             acc_ref, temp_ref, final_ref,
             rs_send, rs_recv, ag_send, ag_recv):
        d = lax.axis_index("i")

        bsem = pltpu.get_barrier_semaphore()
        for o in range(1, N_DEV):
            pl.semaphore_signal(
                bsem, inc=1,
                device_id=(jnp.mod(d + o, N_DEV),),
                device_id_type=pl.DeviceIdType.MESH,
            )

        xv = x_ref[:, :, :]
        kv = k_ref[:, :]
        conv = xv * kv[3:4, :][None, :, :]
        for t in range(3):
            sh = 3 - t
            shifted = jnp.concatenate(
                [jnp.zeros((B, sh, C), jnp.float32), xv[:, : S - sh, :]],
                axis=1,
            )
            conv += shifted * kv[t:t + 1, :][None, :, :]
        a = conv * jax.nn.sigmoid(conv)
        partial = jnp.dot(
            a.reshape(ROWS, C).astype(jnp.bfloat16),
            w_ref[:, :].astype(jnp.bfloat16),
            preferred_element_type=jnp.float32,
        )
        acc_ref[...] = partial.reshape(N_DEV, CHUNK, H).astype(jnp.bfloat16)

        pl.semaphore_wait(bsem, N_DEV - 1)

        descs = []
        for o in range(1, N_DEV):
            tgt = jnp.mod(d + o, N_DEV)
            r = pltpu.make_async_remote_copy(
                src_ref=acc_ref.at[tgt],
                dst_ref=temp_ref.at[o],
                send_sem=rs_send.at[o],
                recv_sem=rs_recv.at[o],
                device_id=(tgt,),
                device_id_type=pl.DeviceIdType.MESH,
            )
            r.start()
            descs.append(r)
        for r in descs:
            r.wait()

        own = acc_ref[pl.ds(d, 1), :, :].astype(jnp.float32)
        others = jnp.sum(temp_ref[1:, :, :].astype(jnp.float32), axis=0)
        reduced = (own[0] + others).astype(jnp.bfloat16)
        final_ref[pl.ds(d, 1), :, :] = reduced[None, :, :]

        descs2 = []
        for o in range(1, N_DEV):
            tgt = jnp.mod(d + o, N_DEV)
            r = pltpu.make_async_remote_copy(
                src_ref=final_ref.at[d],
                dst_ref=final_ref.at[d],
                send_sem=ag_send.at[o],
                recv_sem=ag_recv.at[o],
                device_id=(tgt,),
                device_id_type=pl.DeviceIdType.MESH,
            )
            r.start()
            descs2.append(r)
        for r in descs2:
            r.wait()

        out_ref[...] = final_ref[:, :, :].reshape(B, S, H)

    return pl.pallas_call(
        body,
        out_shape=jax.ShapeDtypeStruct((B, S, H), jnp.bfloat16),
        in_specs=[pl.BlockSpec(memory_space=pltpu.VMEM)] * 3,
        out_specs=pl.BlockSpec(memory_space=pltpu.VMEM),
        scratch_shapes=[
            pltpu.VMEM((N_DEV, CHUNK, H), jnp.bfloat16),
            pltpu.VMEM((N_DEV, CHUNK, H), jnp.bfloat16),
            pltpu.VMEM((N_DEV, CHUNK, H), jnp.bfloat16),
            pltpu.SemaphoreType.DMA((N_DEV,)),
            pltpu.SemaphoreType.DMA((N_DEV,)),
            pltpu.SemaphoreType.DMA((N_DEV,)),
            pltpu.SemaphoreType.DMA((N_DEV,)),
        ],
        compiler_params=pltpu.CompilerParams(collective_id=0),
    )(x, k, Wp)


# device time: 31823 ns/iter; 1.0754x vs baseline; 1.0177x over previous
import jax
import jax.numpy as jnp
from jax import lax
from jax.experimental import pallas as pl
from jax.experimental.pallas import tpu as pltpu

N_DEV = 16
B, S, C, H = 4, 512, 256, 256
ROWS = B * S
CHUNK = ROWS // N_DEV


def kernel(x, k, Wp):
    def body(x_ref, k_ref, w_ref, out_ref,
             acc_ref, temp_ref, final_ref,
             rs_send, rs_recv, ag_send, ag_recv):
        d = lax.axis_index("i")

        bsem = pltpu.get_barrier_semaphore()
        for o in range(1, N_DEV):
            pl.semaphore_signal(
                bsem, inc=1,
                device_id=(jnp.mod(d + o, N_DEV),),
                device_id_type=pl.DeviceIdType.MESH,
            )

        xv = x_ref[:, :, :]
        kv = k_ref[:, :]
        conv = xv * kv[3:4, :][None, :, :]
        for t in range(3):
            sh = 3 - t
            shifted = jnp.concatenate(
                [jnp.zeros((B, sh, C), jnp.float32), xv[:, : S - sh, :]],
                axis=1,
            )
            conv += shifted * kv[t:t + 1, :][None, :, :]
        a = conv * jax.nn.sigmoid(conv)
        partial = jnp.dot(
            a.reshape(ROWS, C).astype(jnp.bfloat16),
            w_ref[:, :].astype(jnp.bfloat16),
            preferred_element_type=jnp.float32,
        )
        acc_ref[...] = partial.reshape(N_DEV, CHUNK, H).astype(jnp.bfloat16)

        pl.semaphore_wait(bsem, N_DEV - 1)

        descs = []
        for o in range(1, N_DEV):
            tgt = jnp.mod(d + o, N_DEV)
            r = pltpu.make_async_remote_copy(
                src_ref=acc_ref.at[tgt],
                dst_ref=temp_ref.at[o],
                send_sem=rs_send.at[o],
                recv_sem=rs_recv.at[o],
                device_id=(tgt,),
                device_id_type=pl.DeviceIdType.MESH,
            )
            r.start()
            descs.append(r)

        red = acc_ref[pl.ds(d, 1), :, :].astype(jnp.float32)[0]
        for o in range(1, N_DEV):
            descs[o - 1].wait_recv()
            red = red + temp_ref[o, :, :].astype(jnp.float32)
        final_ref[pl.ds(d, 1), :, :] = red.astype(jnp.bfloat16)[None, :, :]

        descs2 = []
        for o in range(1, N_DEV):
            tgt = jnp.mod(d + o, N_DEV)
            r = pltpu.make_async_remote_copy(
                src_ref=final_ref.at[d],
                dst_ref=final_ref.at[d],
                send_sem=ag_send.at[o],
                recv_sem=ag_recv.at[o],
                device_id=(tgt,),
                device_id_type=pl.DeviceIdType.MESH,
            )
            r.start()
            descs2.append(r)
        for r in descs2:
            r.wait_recv()

        out_ref[...] = final_ref[:, :, :].reshape(B, S, H)

        for r in descs:
            r.wait_send()
        for r in descs2:
            r.wait_send()

    return pl.pallas_call(
        body,
        out_shape=jax.ShapeDtypeStruct((B, S, H), jnp.bfloat16),
        in_specs=[pl.BlockSpec(memory_space=pltpu.VMEM)] * 3,
        out_specs=pl.BlockSpec(memory_space=pltpu.VMEM),
        scratch_shapes=[
            pltpu.VMEM((N_DEV, CHUNK, H), jnp.bfloat16),
            pltpu.VMEM((N_DEV, CHUNK, H), jnp.bfloat16),
            pltpu.VMEM((N_DEV, CHUNK, H), jnp.bfloat16),
            pltpu.SemaphoreType.DMA((N_DEV,)),
            pltpu.SemaphoreType.DMA((N_DEV,)),
            pltpu.SemaphoreType.DMA((N_DEV,)),
            pltpu.SemaphoreType.DMA((N_DEV,)),
        ],
        compiler_params=pltpu.CompilerParams(collective_id=0),
    )(x, k, Wp)


# device time: 31689 ns/iter; 1.0799x vs baseline; 1.0042x over previous
import jax
import jax.numpy as jnp
from jax import lax
from jax.experimental import pallas as pl
from jax.experimental.pallas import tpu as pltpu

N_DEV = 16
B, S, C, H = 4, 512, 256, 256
ROWS = B * S
CHUNK = ROWS // N_DEV


def kernel(x, k, Wp):
    def body(x_ref, k_ref, w_ref, out_ref,
             acc_ref, temp_ref,
             rs_send, rs_recv, ag_send, ag_recv):
        d = lax.axis_index("i")
        b_idx = d // (S // CHUNK)
        s0 = jnp.mod(d, S // CHUNK) * CHUNK

        bsem = pltpu.get_barrier_semaphore()
        for o in range(1, N_DEV):
            pl.semaphore_signal(
                bsem, inc=1,
                device_id=(jnp.mod(d + o, N_DEV),),
                device_id_type=pl.DeviceIdType.MESH,
            )

        xv = x_ref[:, :, :]
        kv = k_ref[:, :]
        conv = xv * kv[3:4, :][None, :, :]
        for t in range(3):
            sh = 3 - t
            shifted = jnp.concatenate(
                [jnp.zeros((B, sh, C), jnp.float32), xv[:, : S - sh, :]],
                axis=1,
            )
            conv += shifted * kv[t:t + 1, :][None, :, :]
        a = conv * jax.nn.sigmoid(conv)
        partial = jnp.dot(
            a.reshape(ROWS, C).astype(jnp.bfloat16),
            w_ref[:, :].astype(jnp.bfloat16),
            preferred_element_type=jnp.float32,
        )
        acc_ref[...] = partial.reshape(N_DEV, CHUNK, H).astype(jnp.bfloat16)

        pl.semaphore_wait(bsem, N_DEV - 1)

        descs = []
        for o in range(1, N_DEV):
            tgt = jnp.mod(d + o, N_DEV)
            r = pltpu.make_async_remote_copy(
                src_ref=acc_ref.at[tgt],
                dst_ref=temp_ref.at[o],
                send_sem=rs_send.at[o],
                recv_sem=rs_recv.at[o],
                device_id=(tgt,),
                device_id_type=pl.DeviceIdType.MESH,
            )
            r.start()
            descs.append(r)

        red = acc_ref[pl.ds(d, 1), :, :].astype(jnp.float32)[0]
        for o in range(1, N_DEV):
            descs[o - 1].wait_recv()
            red = red + temp_ref[o, :, :].astype(jnp.float32)
        out_ref[b_idx, pl.ds(s0, CHUNK), :] = red.astype(jnp.bfloat16)

        descs2 = []
        for o in range(1, N_DEV):
            tgt = jnp.mod(d + o, N_DEV)
            r = pltpu.make_async_remote_copy(
                src_ref=out_ref.at[b_idx, pl.ds(s0, CHUNK)],
                dst_ref=out_ref.at[b_idx, pl.ds(s0, CHUNK)],
                send_sem=ag_send.at[o],
                recv_sem=ag_recv.at[o],
                device_id=(tgt,),
                device_id_type=pl.DeviceIdType.MESH,
            )
            r.start()
            descs2.append(r)
        for r in descs2:
            r.wait_recv()

        for r in descs:
            r.wait_send()
        for r in descs2:
            r.wait_send()

    return pl.pallas_call(
        body,
        out_shape=jax.ShapeDtypeStruct((B, S, H), jnp.bfloat16),
        in_specs=[pl.BlockSpec(memory_space=pltpu.VMEM)] * 3,
        out_specs=pl.BlockSpec(memory_space=pltpu.VMEM),
        scratch_shapes=[
            pltpu.VMEM((N_DEV, CHUNK, H), jnp.bfloat16),
            pltpu.VMEM((N_DEV, CHUNK, H), jnp.bfloat16),
            pltpu.SemaphoreType.DMA((N_DEV,)),
            pltpu.SemaphoreType.DMA((N_DEV,)),
            pltpu.SemaphoreType.DMA((N_DEV,)),
            pltpu.SemaphoreType.DMA((N_DEV,)),
        ],
        compiler_params=pltpu.CompilerParams(collective_id=0),
    )(x, k, Wp)
